# baseline (speedup 1.0000x reference)
import jax
import jax.numpy as jnp
from jax import lax
from jax.experimental import pallas as pl
from jax.experimental.pallas import tpu as pltpu

TILE_M = 4096


def _proj_kernel(xt_ref, wt_ref, b_ref, o_ref):
    o_ref[...] = (
        lax.dot_general(
            xt_ref[...],
            wt_ref[...],
            dimension_numbers=(((0,), (0,)), ((), ())),
            preferred_element_type=jnp.float32,
        )
        + b_ref[...]
    )


def kernel(latents, W, b):
    B, S, K = latents.shape
    E = W.shape[0]
    M = B * S
    xt = latents.reshape(M, K).T
    wt = W.T
    b2 = b.reshape(1, E)
    grid = (M // TILE_M,)
    out = pl.pallas_call(
        _proj_kernel,
        grid=grid,
        in_specs=[
            pl.BlockSpec((K, TILE_M), lambda i: (0, i)),
            pl.BlockSpec((K, E), lambda i: (0, 0)),
            pl.BlockSpec((1, E), lambda i: (0, 0)),
        ],
        out_specs=pl.BlockSpec((TILE_M, E), lambda i: (i, 0)),
        out_shape=jax.ShapeDtypeStruct((M, E), jnp.float32),
        compiler_params=pltpu.CompilerParams(
            dimension_semantics=("parallel",),
        ),
    )(xt, wt, b2)
    return out.reshape(B, S, E)


# speedup vs baseline: 1.0395x; 1.0395x over previous
import jax
import jax.numpy as jnp
from jax import lax
from jax.experimental import pallas as pl
from jax.experimental.pallas import tpu as pltpu

TILE_M = 1024


def _proj_kernel(xt_ref, wt_ref, b_ref, o_ref):
    o_ref[...] = (
        lax.dot_general(
            xt_ref[...],
            wt_ref[...],
            dimension_numbers=(((0,), (0,)), ((), ())),
            preferred_element_type=jnp.float32,
        )
        + b_ref[...]
    )


def kernel(latents, W, b):
    B, S, K = latents.shape
    E = W.shape[0]
    M = B * S
    xt = latents.reshape(M, K).T
    wt = W.T
    b2 = b.reshape(1, E)
    grid = (M // TILE_M,)
    out = pl.pallas_call(
        _proj_kernel,
        grid=grid,
        in_specs=[
            pl.BlockSpec((K, TILE_M), lambda i: (0, i)),
            pl.BlockSpec((K, E), lambda i: (0, 0)),
            pl.BlockSpec((1, E), lambda i: (0, 0)),
        ],
        out_specs=pl.BlockSpec((TILE_M, E), lambda i: (i, 0)),
        out_shape=jax.ShapeDtypeStruct((M, E), jnp.float32),
        compiler_params=pltpu.CompilerParams(
            dimension_semantics=("parallel",),
        ),
    )(xt, wt, b2)
    return out.reshape(B, S, E)


# speedup vs baseline: 1.0412x; 1.0016x over previous
import jax
import jax.numpy as jnp
from jax import lax
from jax.experimental import pallas as pl
from jax.experimental.pallas import tpu as pltpu

TILE_M = 2048


def _proj_kernel(xt_ref, wt_ref, b_ref, o_ref):
    o_ref[...] = (
        lax.dot_general(
            xt_ref[...],
            wt_ref[...],
            dimension_numbers=(((0,), (0,)), ((), ())),
            preferred_element_type=jnp.float32,
        )
        + b_ref[...]
    )


def kernel(latents, W, b):
    B, S, K = latents.shape
    E = W.shape[0]
    M = B * S
    xt = latents.reshape(M, K).T
    wt = W.T
    b2 = b.reshape(1, E)
    grid = (M // TILE_M,)
    out = pl.pallas_call(
        _proj_kernel,
        grid=grid,
        in_specs=[
            pl.BlockSpec((K, TILE_M), lambda i: (0, i)),
            pl.BlockSpec((K, E), lambda i: (0, 0)),
            pl.BlockSpec((1, E), lambda i: (0, 0)),
        ],
        out_specs=pl.BlockSpec((TILE_M, E), lambda i: (i, 0)),
        out_shape=jax.ShapeDtypeStruct((M, E), jnp.float32),
        compiler_params=pltpu.CompilerParams(
            dimension_semantics=("parallel",),
        ),
    )(xt, wt, b2)
    return out.reshape(B, S, E)
